# baseline (speedup 1.0000x reference)
import functools

import jax
import jax.numpy as jnp
from jax import lax
from jax.experimental import pallas as pl
from jax.experimental.pallas import tpu as pltpu
from jax.experimental.pallas import tpu_sc as plsc

D_MODEL = 1024
SCALE = 32.0

NC = 2
NS = 16
NW = NC * NS
LANES = 16

X_ROWS = 4
X_COLS = 2048
B_TOTAL = X_ROWS * X_COLS
RPW = B_TOTAL // NW
WPR = X_COLS // RPW
CHUNK = 16
NCHUNK = RPW // CHUNK
NBUF = 7
LOOKAHEAD = 4


def _make_kernel():
    mesh = plsc.VectorSubcoreMesh(core_axis_name="c", subcore_axis_name="s")

    @functools.partial(
        pl.kernel,
        mesh=mesh,
        out_type=jax.ShapeDtypeStruct((B_TOTAL, D_MODEL), jnp.float32),
        scratch_types=(
            [pltpu.VMEM((NCHUNK, CHUNK), jnp.int32)]
            + [pltpu.VMEM((CHUNK, D_MODEL), jnp.float32)] * NBUF
            + [pltpu.SemaphoreType.DMA] * (2 * NBUF)
        ),
    )
    def emb_kernel(x_hbm, table_hbm, out_hbm, idx_v,
                   b0, b1, b2, b3, b4, b5, b6,
                   si0, si1, si2, si3, si4, si5, si6,
                   so0, so1, so2, so3, so4, so5, so6):
        wid = lax.axis_index("s") * NC + lax.axis_index("c")
        base = wid * RPW
        xr = wid // WPR
        xc = (wid % WPR) * RPW
        for j in range(NCHUNK):
            pltpu.sync_copy(
                x_hbm.at[xr, pl.ds(xc + j * CHUNK, CHUNK)], idx_v.at[j])

        bufs = (b0, b1, b2, b3, b4, b5, b6)
        sins = (si0, si1, si2, si3, si4, si5, si6)
        souts = (so0, so1, so2, so3, so4, so5, so6)
        gathers = [None] * NBUF
        outs = [None] * NBUF

        def start_gather(j):
            p = j % NBUF
            gathers[p] = pltpu.async_copy(
                table_hbm.at[idx_v.at[j]], bufs[p], sins[p])

        for j in range(min(LOOKAHEAD, NCHUNK)):
            start_gather(j)

        for g in range(NCHUNK):
            p = g % NBUF
            j = g + LOOKAHEAD
            if j < NCHUNK:
                if j - NBUF >= 0 and outs[j % NBUF] is not None:
                    outs[j % NBUF].wait()
                    outs[j % NBUF] = None
                start_gather(j)
            gathers[p].wait()

            buf = bufs[p]

            def scale_row(r, carry, buf=buf):
                for col in range(D_MODEL // LANES):
                    sl = pl.ds(col * LANES, LANES)
                    buf[r, sl] = buf[r, sl] * SCALE
                return carry

            lax.fori_loop(0, CHUNK, scale_row, 0)

            outs[p] = pltpu.async_copy(
                buf, out_hbm.at[pl.ds(base + g * CHUNK, CHUNK)], souts[p])

        for p in range(NBUF):
            if outs[p] is not None:
                outs[p].wait()
                outs[p] = None

    return emb_kernel


_emb_kernel = _make_kernel()


def kernel(x, table):
    out = _emb_kernel(x.astype(jnp.int32), table)
    return out.reshape(x.shape + (D_MODEL,))


# speedup vs baseline: 1.1233x; 1.1233x over previous
import functools

import jax
import jax.numpy as jnp
from jax import lax
from jax.experimental import pallas as pl
from jax.experimental.pallas import tpu as pltpu
from jax.experimental.pallas import tpu_sc as plsc

D_MODEL = 1024
SCALE = 32.0

NC = 2
NS = 16
NW = NC * NS
LANES = 16

X_ROWS = 4
X_COLS = 2048
B_TOTAL = X_ROWS * X_COLS
RPW = B_TOTAL // NW
WPR = X_COLS // RPW
CHUNK = 16
NCHUNK = RPW // CHUNK
NBUF = 7
LOOKAHEAD = 4


def _make_kernel():
    mesh = plsc.VectorSubcoreMesh(core_axis_name="c", subcore_axis_name="s")

    @functools.partial(
        pl.kernel,
        mesh=mesh,
        out_type=jax.ShapeDtypeStruct((B_TOTAL, D_MODEL), jnp.float32),
        scratch_types=(
            [pltpu.VMEM((NCHUNK, CHUNK), jnp.int32)]
            + [pltpu.VMEM((CHUNK, D_MODEL), jnp.float32)] * NBUF
            + [pltpu.SemaphoreType.DMA] * (2 * NBUF)
        ),
    )
    def emb_kernel(x_hbm, table_hbm, out_hbm, idx_v,
                   b0, b1, b2, b3, b4, b5, b6,
                   si0, si1, si2, si3, si4, si5, si6,
                   so0, so1, so2, so3, so4, so5, so6):
        wid = lax.axis_index("s") * NC + lax.axis_index("c")
        base = wid * RPW
        pltpu.sync_copy(x_hbm.at[wid], idx_v)

        bufs = (b0, b1, b2, b3, b4, b5, b6)
        sins = (si0, si1, si2, si3, si4, si5, si6)
        souts = (so0, so1, so2, so3, so4, so5, so6)
        gathers = [None] * NBUF
        outs = [None] * NBUF

        def start_gather(j):
            p = j % NBUF
            gathers[p] = pltpu.async_copy(
                table_hbm.at[idx_v.at[j]], bufs[p], sins[p])

        for j in range(min(LOOKAHEAD, NCHUNK)):
            start_gather(j)

        for g in range(NCHUNK):
            p = g % NBUF
            j = g + LOOKAHEAD
            if j < NCHUNK:
                if j - NBUF >= 0 and outs[j % NBUF] is not None:
                    outs[j % NBUF].wait()
                    outs[j % NBUF] = None
                start_gather(j)
            gathers[p].wait()

            buf = bufs[p]

            def scale_row(r, carry, buf=buf):
                for col in range(D_MODEL // LANES):
                    sl = pl.ds(col * LANES, LANES)
                    buf[r, sl] = buf[r, sl] * SCALE
                return carry

            lax.fori_loop(0, CHUNK, scale_row, 0)

            outs[p] = pltpu.async_copy(
                buf, out_hbm.at[pl.ds(base + g * CHUNK, CHUNK)], souts[p])

        for p in range(NBUF):
            if outs[p] is not None:
                outs[p].wait()
                outs[p] = None

    return emb_kernel


_emb_kernel = _make_kernel()


def kernel(x, table):
    idx = x.astype(jnp.int32).reshape(NW, NCHUNK, CHUNK)
    out = _emb_kernel(idx, table)
    return out.reshape(x.shape + (D_MODEL,))


# speedup vs baseline: 1.1333x; 1.0089x over previous
import functools

import jax
import jax.numpy as jnp
from jax import lax
from jax.experimental import pallas as pl
from jax.experimental.pallas import tpu as pltpu
from jax.experimental.pallas import tpu_sc as plsc

D_MODEL = 1024
SCALE = 32.0

NC = 2
NS = 16
NW = NC * NS
LANES = 16

X_ROWS = 4
X_COLS = 2048
B_TOTAL = X_ROWS * X_COLS
RPW = B_TOTAL // NW
WPR = X_COLS // RPW
CHUNK = 16
NCHUNK = RPW // CHUNK
NBUF = 7
LOOKAHEAD = 5


def _make_kernel():
    mesh = plsc.VectorSubcoreMesh(core_axis_name="c", subcore_axis_name="s")

    @functools.partial(
        pl.kernel,
        mesh=mesh,
        out_type=jax.ShapeDtypeStruct((B_TOTAL, D_MODEL), jnp.float32),
        scratch_types=(
            [pltpu.VMEM((NCHUNK, CHUNK), jnp.int32)]
            + [pltpu.VMEM((CHUNK, D_MODEL), jnp.float32)] * NBUF
            + [pltpu.SemaphoreType.DMA] * (2 * NBUF)
        ),
    )
    def emb_kernel(x_hbm, table_hbm, out_hbm, idx_v,
                   b0, b1, b2, b3, b4, b5, b6,
                   si0, si1, si2, si3, si4, si5, si6,
                   so0, so1, so2, so3, so4, so5, so6):
        wid = lax.axis_index("s") * NC + lax.axis_index("c")
        base = wid * RPW
        pltpu.sync_copy(x_hbm.at[wid], idx_v)

        bufs = (b0, b1, b2, b3, b4, b5, b6)
        sins = (si0, si1, si2, si3, si4, si5, si6)
        souts = (so0, so1, so2, so3, so4, so5, so6)
        gathers = [None] * NBUF
        outs = [None] * NBUF

        def start_gather(j):
            p = j % NBUF
            gathers[p] = pltpu.async_copy(
                table_hbm.at[idx_v.at[j]], bufs[p], sins[p])

        for j in range(min(LOOKAHEAD, NCHUNK)):
            start_gather(j)

        for g in range(NCHUNK):
            p = g % NBUF
            j = g + LOOKAHEAD
            if j < NCHUNK:
                if j - NBUF >= 0 and outs[j % NBUF] is not None:
                    outs[j % NBUF].wait()
                    outs[j % NBUF] = None
                start_gather(j)
            gathers[p].wait()

            buf = bufs[p]

            def scale_row(r, carry, buf=buf):
                for col in range(D_MODEL // LANES):
                    sl = pl.ds(col * LANES, LANES)
                    buf[r, sl] = buf[r, sl] * SCALE
                return carry

            lax.fori_loop(0, CHUNK, scale_row, 0)

            outs[p] = pltpu.async_copy(
                buf, out_hbm.at[pl.ds(base + g * CHUNK, CHUNK)], souts[p])

        for p in range(NBUF):
            if outs[p] is not None:
                outs[p].wait()
                outs[p] = None

    return emb_kernel


_emb_kernel = _make_kernel()


def kernel(x, table):
    idx = x.astype(jnp.int32).reshape(NW, NCHUNK, CHUNK)
    out = _emb_kernel(idx, table)
    return out.reshape(x.shape + (D_MODEL,))
